# baseline (speedup 1.0000x reference)
import functools

import jax
import jax.numpy as jnp
from jax import lax
from jax.experimental import pallas as pl
from jax.experimental.pallas import tpu as pltpu
from jax.experimental.pallas import tpu_sc as plsc

D = 64
NC = 2
NS = 16
NW = NC * NS
BB = 128
CT = D // 8


@functools.partial(jax.jit, static_argnames=("h", "nbt"))
def _gather(table2, ids_lm, h, nbt):
    n_units = h * nbt
    upw = n_units // NW
    mesh = plsc.VectorSubcoreMesh(core_axis_name="c", subcore_axis_name="s")

    @functools.partial(
        pl.kernel,
        mesh=mesh,
        out_type=jax.ShapeDtypeStruct((h * CT * nbt, 8 * BB), jnp.float32),
        scratch_types=[
            pltpu.VMEM((2, BB), jnp.int32),
            pltpu.VMEM((2, BB), jnp.int32),
            pltpu.VMEM((2, BB), jnp.int32),
            pltpu.VMEM((2, BB, 128), jnp.float32),
            pltpu.VMEM((2, D * BB), jnp.float32),
            pltpu.SemaphoreType.DMA,
            pltpu.SemaphoreType.DMA,
            pltpu.SemaphoreType.DMA,
            pltpu.SemaphoreType.DMA,
        ],
        compiler_params=pltpu.CompilerParams(use_tc_tiling_on_sc=False,
                                             needs_layout_passes=False),
    )
    def k(tab_hbm, idx_hbm, out_hbm, idx_v, pidx_v, hoff_v, prows_v, y_v,
          g0, g1, o0, o1):
        gsem = (g0, g1)
        osem = (o0, o1)
        wid = lax.axis_index("s") * NC + lax.axis_index("c")
        u_base = wid * upw
        iota = lax.iota(jnp.int32, 16)

        def unit_lbt(u):
            ug = u_base + u
            return ug // nbt, lax.rem(ug, nbt)

        def stage_a(u, q):
            l, bt = unit_lbt(u)
            pltpu.sync_copy(idx_hbm.at[pl.ds(l * (nbt * BB) + bt * BB, BB)],
                            idx_v.at[q])
            for g in range(BB // 16):
                v = idx_v[q, pl.ds(g * 16, 16)]
                pidx_v[q, pl.ds(g * 16, 16)] = lax.shift_right_logical(v, 1)
                hoff_v[q, pl.ds(g * 16, 16)] = (
                    lax.shift_left(lax.bitwise_and(v, 1), 6))
            pltpu.async_copy(tab_hbm.at[pidx_v.at[q]], prows_v.at[q], gsem[q])

        def wait_gather(q):
            pltpu.make_async_copy(tab_hbm.at[pidx_v.at[q]], prows_v.at[q],
                                  gsem[q]).wait()

        def fire_out(u, q):
            l, bt = unit_lbt(u)
            for ct in range(CT):
                pltpu.async_copy(
                    y_v.at[q, pl.ds(ct * (8 * BB), 8 * BB)],
                    out_hbm.at[(l * CT + ct) * nbt + bt],
                    osem[q],
                )

        def wait_out(q):
            for ct in range(CT):
                pltpu.make_async_copy(
                    y_v.at[q, pl.ds(ct * (8 * BB), 8 * BB)],
                    out_hbm.at[0], osem[q]).wait()

        def transpose(q):
            def sb_body(sb, carry):
                g = lax.shift_right_logical(sb, 2)
                cb = lax.bitwise_and(sb, 3)
                hv = hoff_v[q, pl.ds(g * 16, 16)]
                bi_vec = iota + g * 16
                col0 = hv + cb * 16
                svec = iota + (cb * (16 * BB) + g * 16)
                for j in range(16):
                    rot = lax.bitwise_and(iota + j, 15)
                    val = plsc.load_gather(prows_v.at[q],
                                           [bi_vec, col0 + rot])
                    plsc.store_scatter(y_v.at[q],
                                       [svec + lax.shift_left(rot, 7)], val)
                return carry

            lax.fori_loop(0, (BB // 16) * (D // 16), sb_body, 0)

        def stage_b(u, q):
            wait_gather(q)
            wait_out(q)
            transpose(q)
            fire_out(u, q)

        stage_a(0, 0)
        stage_a(1, 1)
        fire_out(0, 0)
        fire_out(1, 1)

        def body2(kk, carry):
            u = 2 * kk
            stage_b(u, 0)
            stage_a(u + 2, 0)
            stage_b(u + 1, 1)
            stage_a(u + 3, 1)
            return carry

        lax.fori_loop(0, (upw - 2) // 2, body2, 0)

        stage_b(upw - 2, 0)
        stage_b(upw - 1, 1)
        wait_out(0)
        wait_out(1)

    return k(table2, ids_lm)


def kernel(ids, table):
    b, h = ids.shape
    v, d = table.shape
    assert d == D and v % 2 == 0 and b % BB == 0
    nbt = b // BB
    assert (h * nbt) % (2 * NW) == 0
    table2 = lax.optimization_barrier(table.reshape(v // 2, 2 * D))
    ids_lm = ids.T.reshape(-1).astype(jnp.int32)
    y = _gather(table2, ids_lm, h, nbt)
    return (y.reshape(h, CT, nbt, 8, BB)
             .transpose(2, 4, 0, 1, 3)
             .reshape(b, h, D))


# speedup vs baseline: 1.0910x; 1.0910x over previous
import functools

import jax
import jax.numpy as jnp
from jax import lax
from jax.experimental import pallas as pl
from jax.experimental.pallas import tpu as pltpu
from jax.experimental.pallas import tpu_sc as plsc

D = 64
NC = 2
NS = 16
NW = NC * NS
BB = 128
CT = D // 8


@functools.partial(jax.jit, static_argnames=("h", "nbt"))
def _gather(table2, ids_lm, h, nbt):
    n_units = h * nbt
    upw = n_units // NW
    mesh = plsc.VectorSubcoreMesh(core_axis_name="c", subcore_axis_name="s")

    @functools.partial(
        pl.kernel,
        mesh=mesh,
        out_type=jax.ShapeDtypeStruct((h * CT * nbt, 8 * BB), jnp.float32),
        scratch_types=[
            pltpu.VMEM((2, BB), jnp.int32),
            pltpu.VMEM((2, BB), jnp.int32),
            pltpu.VMEM((2, BB), jnp.int32),
            pltpu.VMEM((2, BB, 128), jnp.float32),
            pltpu.VMEM((2, D * BB), jnp.float32),
            pltpu.SemaphoreType.DMA,
            pltpu.SemaphoreType.DMA,
            pltpu.SemaphoreType.DMA,
            pltpu.SemaphoreType.DMA,
            pltpu.SemaphoreType.DMA,
            pltpu.SemaphoreType.DMA,
        ],
        compiler_params=pltpu.CompilerParams(use_tc_tiling_on_sc=False,
                                             needs_layout_passes=False),
    )
    def k(tab_hbm, idx_hbm, out_hbm, idx_v, pidx_v, hoff_v, prows_v, y_v,
          g0, g1, o0, o1, i0, i1):
        gsem = (g0, g1)
        osem = (o0, o1)
        isem = (i0, i1)
        wid = lax.axis_index("s") * NC + lax.axis_index("c")
        u_base = wid * upw
        iota = lax.iota(jnp.int32, 16)

        def unit_lbt(u):
            ug = u_base + u
            return ug // nbt, lax.rem(ug, nbt)

        def fire_idx(u, q):
            l, bt = unit_lbt(u)
            pltpu.async_copy(idx_hbm.at[pl.ds(l * (nbt * BB) + bt * BB, BB)],
                             idx_v.at[q], isem[q])

        def stage_af(u, q):
            pltpu.make_async_copy(idx_hbm.at[pl.ds(0, BB)], idx_v.at[q],
                                  isem[q]).wait()
            for g in range(BB // 16):
                v = idx_v[q, pl.ds(g * 16, 16)]
                pidx_v[q, pl.ds(g * 16, 16)] = lax.shift_right_logical(v, 1)
                hoff_v[q, pl.ds(g * 16, 16)] = (
                    lax.shift_left(lax.bitwise_and(v, 1), 6))
            hb = BB // 2
            for s in range(2):
                pltpu.async_copy(tab_hbm.at[pidx_v.at[q, pl.ds(s * hb, hb)]],
                                 prows_v.at[q, pl.ds(s * hb, hb)], gsem[q])

        def wait_gather(q):
            hb = BB // 2
            for s in range(2):
                pltpu.make_async_copy(
                    tab_hbm.at[pidx_v.at[q, pl.ds(s * hb, hb)]],
                    prows_v.at[q, pl.ds(s * hb, hb)], gsem[q]).wait()

        def fire_out(u, q):
            l, bt = unit_lbt(u)
            for ct in range(CT):
                pltpu.async_copy(
                    y_v.at[q, pl.ds(ct * (8 * BB), 8 * BB)],
                    out_hbm.at[(l * CT + ct) * nbt + bt],
                    osem[q],
                )

        def wait_out(q):
            for ct in range(CT):
                pltpu.make_async_copy(
                    y_v.at[q, pl.ds(ct * (8 * BB), 8 * BB)],
                    out_hbm.at[0], osem[q]).wait()

        def transpose(q):
            def sb_body(sb, carry):
                g = lax.shift_right_logical(sb, 2)
                cb = lax.bitwise_and(sb, 3)
                hv = hoff_v[q, pl.ds(g * 16, 16)]
                bi_vec = iota + g * 16
                col0 = hv + cb * 16
                svec = iota + (cb * (16 * BB) + g * 16)
                for j in range(16):
                    rot = lax.bitwise_and(iota + j, 15)
                    val = plsc.load_gather(prows_v.at[q],
                                           [bi_vec, col0 + rot])
                    plsc.store_scatter(y_v.at[q],
                                       [svec + lax.shift_left(rot, 7)], val)
                return carry

            lax.fori_loop(0, (BB // 16) * (D // 16), sb_body, 0)

        def stage_b(u, q):
            wait_gather(q)
            wait_out(q)
            transpose(q)
            fire_out(u, q)

        fire_idx(0, 0)
        fire_idx(1, 1)
        stage_af(0, 0)
        fire_idx(2, 0)
        stage_af(1, 1)
        fire_idx(3, 1)
        fire_out(0, 0)
        fire_out(1, 1)

        def body2(kk, carry):
            u = 2 * kk
            stage_b(u, 0)
            stage_af(u + 2, 0)
            fire_idx(u + 4, 0)
            stage_b(u + 1, 1)
            stage_af(u + 3, 1)
            fire_idx(u + 5, 1)
            return carry

        lax.fori_loop(0, (upw - 6) // 2, body2, 0)

        stage_b(upw - 6, 0)
        stage_af(upw - 4, 0)
        fire_idx(upw - 2, 0)
        stage_b(upw - 5, 1)
        stage_af(upw - 3, 1)
        fire_idx(upw - 1, 1)
        stage_b(upw - 4, 0)
        stage_af(upw - 2, 0)
        stage_b(upw - 3, 1)
        stage_af(upw - 1, 1)
        stage_b(upw - 2, 0)
        stage_b(upw - 1, 1)
        wait_out(0)
        wait_out(1)

    return k(table2, ids_lm)


def kernel(ids, table):
    b, h = ids.shape
    v, d = table.shape
    assert d == D and v % 2 == 0 and b % BB == 0
    nbt = b // BB
    assert (h * nbt) % (2 * NW) == 0
    table2 = lax.optimization_barrier(table.reshape(v // 2, 2 * D))
    ids_lm = ids.T.reshape(-1).astype(jnp.int32)
    y = _gather(table2, ids_lm, h, nbt)
    return (y.reshape(h, CT, nbt, 8, BB)
             .transpose(2, 4, 0, 1, 3)
             .reshape(b, h, D))


# speedup vs baseline: 1.1569x; 1.0604x over previous
import functools

import jax
import jax.numpy as jnp
from jax import lax
from jax.experimental import pallas as pl
from jax.experimental.pallas import tpu as pltpu
from jax.experimental.pallas import tpu_sc as plsc

D = 64
NC = 2
NS = 16
NW = NC * NS
BB = 128
CT = D // 8


@functools.partial(jax.jit, static_argnames=("h", "nbt"))
def _gather(table2, ids_lm, h, nbt):
    n_units = h * nbt
    upw = n_units // NW
    mesh = plsc.VectorSubcoreMesh(core_axis_name="c", subcore_axis_name="s")

    @functools.partial(
        pl.kernel,
        mesh=mesh,
        out_type=jax.ShapeDtypeStruct((h * CT * nbt, 8 * BB), jnp.float32),
        scratch_types=[
            pltpu.VMEM((4, BB), jnp.int32),
            pltpu.VMEM((4, BB), jnp.int32),
            pltpu.VMEM((4, BB, D), jnp.float32),
            pltpu.VMEM((2, D * BB), jnp.float32),
            pltpu.SemaphoreType.DMA,
            pltpu.SemaphoreType.DMA,
            pltpu.SemaphoreType.DMA,
            pltpu.SemaphoreType.DMA,
            pltpu.SemaphoreType.DMA,
            pltpu.SemaphoreType.DMA,
            pltpu.SemaphoreType.DMA,
            pltpu.SemaphoreType.DMA,
            pltpu.SemaphoreType.DMA,
            pltpu.SemaphoreType.DMA,
        ],
        compiler_params=pltpu.CompilerParams(use_tc_tiling_on_sc=False,
                                             needs_layout_passes=False),
    )
    def k(tab_hbm, idx_hbm, out_hbm, idx_v, gidx_v, prows_v, y_v,
          g0, g1, g2, g3, o0, o1, i0, i1, i2, i3):
        gsem = (g0, g1, g2, g3)
        osem = (o0, o1)
        isem = (i0, i1, i2, i3)
        wid = lax.axis_index("s") * NC + lax.axis_index("c")
        u_base = wid * upw
        iota = lax.iota(jnp.int32, 16)

        def unit_lbt(u):
            ug = u_base + u
            return ug // nbt, lax.rem(ug, nbt)

        def fire_idx(u, p):
            l, bt = unit_lbt(u)
            pltpu.async_copy(idx_hbm.at[pl.ds(l * (nbt * BB) + bt * BB, BB)],
                             idx_v.at[p], isem[p])

        def stage_af(u, p):
            pltpu.make_async_copy(idx_hbm.at[pl.ds(0, BB)], idx_v.at[p],
                                  isem[p]).wait()
            for g in range(BB // 16):
                gidx_v[p, pl.ds(g * 16, 16)] = idx_v[p, pl.ds(g * 16, 16)]
            hb = BB // 2
            for s in range(2):
                pltpu.async_copy(tab_hbm.at[gidx_v.at[p, pl.ds(s * hb, hb)]],
                                 prows_v.at[p, pl.ds(s * hb, hb)], gsem[p])

        def wait_gather(p):
            hb = BB // 2
            for s in range(2):
                pltpu.make_async_copy(
                    tab_hbm.at[gidx_v.at[p, pl.ds(s * hb, hb)]],
                    prows_v.at[p, pl.ds(s * hb, hb)], gsem[p]).wait()

        def fire_out(u, q):
            l, bt = unit_lbt(u)
            for ct in range(CT):
                pltpu.async_copy(
                    y_v.at[q, pl.ds(ct * (8 * BB), 8 * BB)],
                    out_hbm.at[(l * CT + ct) * nbt + bt],
                    osem[q],
                )

        def wait_out(q):
            for ct in range(CT):
                pltpu.make_async_copy(
                    y_v.at[q, pl.ds(ct * (8 * BB), 8 * BB)],
                    out_hbm.at[0], osem[q]).wait()

        def transpose(p):
            q = p & 1
            def sb_body(sb, carry):
                g = lax.shift_right_logical(sb, 2)
                cb = lax.bitwise_and(sb, 3)
                bi_vec = iota + g * 16
                col0 = cb * 16
                svec = iota + (cb * (16 * BB) + g * 16)
                for j in range(16):
                    rot = lax.bitwise_and(iota + j, 15)
                    val = plsc.load_gather(prows_v.at[p],
                                           [bi_vec, col0 + rot])
                    plsc.store_scatter(y_v.at[q],
                                       [svec + lax.shift_left(rot, 7)], val)
                return carry

            lax.fori_loop(0, (BB // 16) * (D // 16), sb_body, 0)

        def stage_b(u, p):
            wait_gather(p)
            wait_out(p & 1)
            transpose(p)
            fire_out(u, p & 1)

        for p in range(4):
            fire_idx(p, p)
        for p in range(4):
            stage_af(p, p)
            fire_idx(p + 4, p)
        fire_out(0, 0)
        fire_out(1, 1)

        def body4(kk, carry):
            u = 4 * kk
            for p in range(4):
                stage_b(u + p, p)
                stage_af(u + p + 4, p)
                fire_idx(u + p + 8, p)
            return carry

        lax.fori_loop(0, (upw - 8) // 4, body4, 0)

        for p in range(4):
            stage_b(upw - 8 + p, p)
            stage_af(upw - 4 + p, p)
        for p in range(4):
            stage_b(upw - 4 + p, p)
        wait_out(0)
        wait_out(1)

    return k(table2, ids_lm)


def kernel(ids, table):
    b, h = ids.shape
    v, d = table.shape
    assert d == D and b % BB == 0
    nbt = b // BB
    assert (h * nbt) % (8 * NW) == 0
    ids_lm = ids.T.reshape(-1).astype(jnp.int32)
    y = _gather(table, ids_lm, h, nbt)
    return (y.reshape(h, CT, nbt, 8, BB)
             .transpose(2, 4, 0, 1, 3)
             .reshape(b, h, D))


# speedup vs baseline: 1.3740x; 1.1877x over previous
import functools

import jax
import jax.numpy as jnp
from jax import lax
from jax.experimental import pallas as pl
from jax.experimental.pallas import tpu as pltpu
from jax.experimental.pallas import tpu_sc as plsc

D = 64
NC = 2
NS = 16
NW = NC * NS
BB = 128
CT = D // 8


@functools.partial(jax.jit, static_argnames=("h", "nbt"))
def _gather(table2, ids_lm, h, nbt):
    n_units = h * nbt
    upw = n_units // NW
    mesh = plsc.VectorSubcoreMesh(core_axis_name="c", subcore_axis_name="s")

    @functools.partial(
        pl.kernel,
        mesh=mesh,
        out_type=jax.ShapeDtypeStruct((h * CT * nbt, 8 * BB), jnp.float32),
        scratch_types=[
            pltpu.VMEM((4, BB), jnp.int32),
            pltpu.VMEM((4, BB), jnp.int32),
            pltpu.VMEM((4, BB, D), jnp.float32),
            pltpu.VMEM((2, D * BB), jnp.float32),
            pltpu.SemaphoreType.DMA,
            pltpu.SemaphoreType.DMA,
            pltpu.SemaphoreType.DMA,
            pltpu.SemaphoreType.DMA,
            pltpu.SemaphoreType.DMA,
            pltpu.SemaphoreType.DMA,
            pltpu.SemaphoreType.DMA,
            pltpu.SemaphoreType.DMA,
            pltpu.SemaphoreType.DMA,
            pltpu.SemaphoreType.DMA,
        ],
        compiler_params=pltpu.CompilerParams(use_tc_tiling_on_sc=False,
                                             needs_layout_passes=False),
    )
    def k(tab_hbm, idx_hbm, out_hbm, idx_v, gidx_v, prows_v, y_v,
          g0, g1, g2, g3, o0, o1, i0, i1, i2, i3):
        gsem = (g0, g1, g2, g3)
        osem = (o0, o1)
        isem = (i0, i1, i2, i3)
        wid = lax.axis_index("s") * NC + lax.axis_index("c")
        u_base = wid * upw
        iota = lax.iota(jnp.int32, 16)

        def unit_lbt(u):
            ug = u_base + u
            return ug // nbt, lax.rem(ug, nbt)

        def fire_idx(u, p):
            l, bt = unit_lbt(u)
            pltpu.async_copy(idx_hbm.at[pl.ds(l * (nbt * BB) + bt * BB, BB)],
                             idx_v.at[p], isem[p])

        def stage_af(u, p):
            pltpu.make_async_copy(idx_hbm.at[pl.ds(0, BB)], idx_v.at[p],
                                  isem[p]).wait()
            for g in range(BB // 16):
                gidx_v[p, pl.ds(g * 16, 16)] = idx_v[p, pl.ds(g * 16, 16)]
            hb = BB // 2
            for s in range(2):
                pltpu.async_copy(tab_hbm.at[gidx_v.at[p, pl.ds(s * hb, hb)]],
                                 prows_v.at[p, pl.ds(s * hb, hb)], gsem[p])

        def wait_gather(p):
            hb = BB // 2
            for s in range(2):
                pltpu.make_async_copy(
                    tab_hbm.at[gidx_v.at[p, pl.ds(s * hb, hb)]],
                    prows_v.at[p, pl.ds(s * hb, hb)], gsem[p]).wait()

        def fire_out(u, q):
            l, bt = unit_lbt(u)
            for ct in range(CT):
                pltpu.async_copy(
                    y_v.at[q, pl.ds(ct * (8 * BB), 8 * BB)],
                    out_hbm.at[(l * CT + ct) * nbt + bt],
                    osem[q],
                )

        def wait_out(q):
            for ct in range(CT):
                pltpu.make_async_copy(
                    y_v.at[q, pl.ds(ct * (8 * BB), 8 * BB)],
                    out_hbm.at[0], osem[q]).wait()

        def transpose(p):
            q = p & 1
            def sb_body(sb, carry):
                g = lax.shift_right_logical(sb, 2)
                cb = lax.bitwise_and(sb, 3)
                bi_vec = iota + g * 16
                col0 = cb * 16
                svec = iota + (cb * (16 * BB) + g * 16)
                for j in range(0, 16, 2):
                    rot0 = lax.bitwise_and(iota + j, 15)
                    rot1 = lax.bitwise_and(iota + (j + 1), 15)
                    val0 = plsc.load_gather(prows_v.at[p],
                                            [bi_vec, col0 + rot0])
                    val1 = plsc.load_gather(prows_v.at[p],
                                            [bi_vec, col0 + rot1])
                    plsc.store_scatter(y_v.at[q],
                                       [svec + lax.shift_left(rot0, 7)], val0)
                    plsc.store_scatter(y_v.at[q],
                                       [svec + lax.shift_left(rot1, 7)], val1)
                return carry

            lax.fori_loop(0, (BB // 16) * (D // 16), sb_body, 0)

        def stage_b(u, p):
            wait_gather(p)
            wait_out(p & 1)
            transpose(p)
            fire_out(u, p & 1)

        for p in range(4):
            fire_idx(p, p)
        for p in range(4):
            stage_af(p, p)
            fire_idx(p + 4, p)
        fire_out(0, 0)
        fire_out(1, 1)

        def body4(kk, carry):
            u = 4 * kk
            for p in range(4):
                stage_b(u + p, p)
                stage_af(u + p + 4, p)
                fire_idx(u + p + 8, p)
            return carry

        lax.fori_loop(0, (upw - 8) // 4, body4, 0)

        for p in range(4):
            stage_b(upw - 8 + p, p)
            stage_af(upw - 4 + p, p)
        for p in range(4):
            stage_b(upw - 4 + p, p)
        wait_out(0)
        wait_out(1)

    return k(table2, ids_lm)


def kernel(ids, table):
    b, h = ids.shape
    v, d = table.shape
    assert d == D and b % BB == 0
    nbt = b // BB
    assert (h * nbt) % (8 * NW) == 0
    ids_lm = ids.T.reshape(-1).astype(jnp.int32)
    y = _gather(table, ids_lm, h, nbt)
    return (y.reshape(h, CT, nbt, 8, BB)
             .transpose(2, 4, 0, 1, 3)
             .reshape(b, h, D))


# speedup vs baseline: 1.5395x; 1.1205x over previous
import functools

import jax
import jax.numpy as jnp
from jax import lax
from jax.experimental import pallas as pl
from jax.experimental.pallas import tpu as pltpu
from jax.experimental.pallas import tpu_sc as plsc

D = 64
NC = 2
NS = 16
NW = NC * NS
BB = 128
CT = D // 8


@functools.partial(jax.jit, static_argnames=("h", "nbt"))
def _gather(table2, ids_lm, h, nbt):
    n_units = h * nbt
    upw = n_units // NW
    mesh = plsc.VectorSubcoreMesh(core_axis_name="c", subcore_axis_name="s")

    @functools.partial(
        pl.kernel,
        mesh=mesh,
        out_type=jax.ShapeDtypeStruct((h * CT * nbt, 8 * BB), jnp.float32),
        scratch_types=[
            pltpu.VMEM((4, BB), jnp.int32),
            pltpu.VMEM((4, BB), jnp.int32),
            pltpu.VMEM((4, BB, D), jnp.float32),
            pltpu.VMEM((2, D * BB), jnp.float32),
            pltpu.SemaphoreType.DMA,
            pltpu.SemaphoreType.DMA,
            pltpu.SemaphoreType.DMA,
            pltpu.SemaphoreType.DMA,
            pltpu.SemaphoreType.DMA,
            pltpu.SemaphoreType.DMA,
            pltpu.SemaphoreType.DMA,
            pltpu.SemaphoreType.DMA,
            pltpu.SemaphoreType.DMA,
            pltpu.SemaphoreType.DMA,
        ],
        compiler_params=pltpu.CompilerParams(use_tc_tiling_on_sc=False,
                                             needs_layout_passes=False),
    )
    def k(tab_hbm, idx_hbm, out_hbm, idx_v, gidx_v, prows_v, y_v,
          g0, g1, g2, g3, o0, o1, i0, i1, i2, i3):
        gsem = (g0, g1, g2, g3)
        osem = (o0, o1)
        isem = (i0, i1, i2, i3)
        wid = lax.axis_index("s") * NC + lax.axis_index("c")
        u_base = wid * upw
        iota = lax.iota(jnp.int32, 16)

        def unit_lbt(u):
            ug = u_base + u
            return ug // nbt, lax.rem(ug, nbt)

        def fire_idx(u, p):
            l, bt = unit_lbt(u)
            pltpu.async_copy(idx_hbm.at[pl.ds(l * (nbt * BB) + bt * BB, BB)],
                             idx_v.at[p], isem[p])

        def stage_af(u, p):
            pltpu.make_async_copy(idx_hbm.at[pl.ds(0, BB)], idx_v.at[p],
                                  isem[p]).wait()
            for g in range(BB // 16):
                gidx_v[p, pl.ds(g * 16, 16)] = idx_v[p, pl.ds(g * 16, 16)]
            hb = BB // 2
            for s in range(2):
                pltpu.async_copy(tab_hbm.at[gidx_v.at[p, pl.ds(s * hb, hb)]],
                                 prows_v.at[p, pl.ds(s * hb, hb)], gsem[p])

        def wait_gather(p):
            hb = BB // 2
            for s in range(2):
                pltpu.make_async_copy(
                    tab_hbm.at[gidx_v.at[p, pl.ds(s * hb, hb)]],
                    prows_v.at[p, pl.ds(s * hb, hb)], gsem[p]).wait()

        def fire_out(u, q):
            l, bt = unit_lbt(u)
            for ct in range(CT):
                pltpu.async_copy(
                    y_v.at[q, pl.ds(ct * (8 * BB), 8 * BB)],
                    out_hbm.at[(l * CT + ct) * nbt + bt],
                    osem[q],
                )

        def wait_out(q):
            for ct in range(CT):
                pltpu.make_async_copy(
                    y_v.at[q, pl.ds(ct * (8 * BB), 8 * BB)],
                    out_hbm.at[0], osem[q]).wait()

        def transpose(p):
            q = p & 1
            def sb_body(sb, carry):
                g = lax.shift_right_logical(sb, 2)
                cb = lax.bitwise_and(sb, 3)
                bi_vec = iota + g * 16
                col0 = cb * 16
                svec = iota + (cb * (16 * BB) + g * 16)
                for j in range(0, 16, 4):
                    rots = [lax.bitwise_and(iota + (j + t), 15)
                            for t in range(4)]
                    vals = [plsc.load_gather(prows_v.at[p],
                                             [bi_vec, col0 + r])
                            for r in rots]
                    for r, v in zip(rots, vals):
                        plsc.store_scatter(y_v.at[q],
                                           [svec + lax.shift_left(r, 7)], v)
                return carry

            lax.fori_loop(0, (BB // 16) * (D // 16), sb_body, 0)

        def stage_b(u, p):
            wait_gather(p)
            wait_out(p & 1)
            transpose(p)
            fire_out(u, p & 1)

        for p in range(4):
            fire_idx(p, p)
        for p in range(4):
            stage_af(p, p)
            fire_idx(p + 4, p)
        fire_out(0, 0)
        fire_out(1, 1)

        def body4(kk, carry):
            u = 4 * kk
            for p in range(4):
                stage_b(u + p, p)
                stage_af(u + p + 4, p)
                fire_idx(u + p + 8, p)
            return carry

        lax.fori_loop(0, (upw - 8) // 4, body4, 0)

        for p in range(4):
            stage_b(upw - 8 + p, p)
            stage_af(upw - 4 + p, p)
        for p in range(4):
            stage_b(upw - 4 + p, p)
        wait_out(0)
        wait_out(1)

    return k(table2, ids_lm)


def kernel(ids, table):
    b, h = ids.shape
    v, d = table.shape
    assert d == D and b % BB == 0
    nbt = b // BB
    assert (h * nbt) % (8 * NW) == 0
    ids_lm = ids.T.reshape(-1).astype(jnp.int32)
    y = _gather(table, ids_lm, h, nbt)
    return (y.reshape(h, CT, nbt, 8, BB)
             .transpose(2, 4, 0, 1, 3)
             .reshape(b, h, D))


# speedup vs baseline: 1.5650x; 1.0165x over previous
import functools

import jax
import jax.numpy as jnp
from jax import lax
from jax.experimental import pallas as pl
from jax.experimental.pallas import tpu as pltpu
from jax.experimental.pallas import tpu_sc as plsc

D = 64
NC = 2
NS = 16
NW = NC * NS
BB = 128
CT = D // 8


@functools.partial(jax.jit, static_argnames=("h", "nbt"))
def _gather(table2, ids_lm, h, nbt):
    n_units = h * nbt
    upw = n_units // NW
    mesh = plsc.VectorSubcoreMesh(core_axis_name="c", subcore_axis_name="s")

    @functools.partial(
        pl.kernel,
        mesh=mesh,
        out_type=jax.ShapeDtypeStruct((h * CT * nbt, 8 * BB), jnp.float32),
        scratch_types=[
            pltpu.VMEM((4, BB), jnp.int32),
            pltpu.VMEM((4, BB), jnp.int32),
            pltpu.VMEM((4, BB, D), jnp.float32),
            pltpu.VMEM((2, D * BB), jnp.float32),
            pltpu.SemaphoreType.DMA,
            pltpu.SemaphoreType.DMA,
            pltpu.SemaphoreType.DMA,
            pltpu.SemaphoreType.DMA,
            pltpu.SemaphoreType.DMA,
            pltpu.SemaphoreType.DMA,
            pltpu.SemaphoreType.DMA,
            pltpu.SemaphoreType.DMA,
            pltpu.SemaphoreType.DMA,
            pltpu.SemaphoreType.DMA,
        ],
        compiler_params=pltpu.CompilerParams(use_tc_tiling_on_sc=False,
                                             needs_layout_passes=False),
    )
    def k(tab_hbm, idx_hbm, out_hbm, idx_v, gidx_v, prows_v, y_v,
          g0, g1, g2, g3, o0, o1, i0, i1, i2, i3):
        gsem = (g0, g1, g2, g3)
        osem = (o0, o1)
        isem = (i0, i1, i2, i3)
        wid = lax.axis_index("s") * NC + lax.axis_index("c")
        u_base = wid * upw
        iota = lax.iota(jnp.int32, 16)

        def unit_lbt(u):
            ug = u_base + u
            return ug // nbt, lax.rem(ug, nbt)

        def fire_idx(u, p):
            l, bt = unit_lbt(u)
            pltpu.async_copy(idx_hbm.at[pl.ds(l * (nbt * BB) + bt * BB, BB)],
                             idx_v.at[p], isem[p])

        def stage_af(u, p):
            pltpu.make_async_copy(idx_hbm.at[pl.ds(0, BB)], idx_v.at[p],
                                  isem[p]).wait()
            for g in range(BB // 16):
                gidx_v[p, pl.ds(g * 16, 16)] = idx_v[p, pl.ds(g * 16, 16)]
            hb = BB // 2
            for s in range(2):
                pltpu.async_copy(tab_hbm.at[gidx_v.at[p, pl.ds(s * hb, hb)]],
                                 prows_v.at[p, pl.ds(s * hb, hb)], gsem[p])

        def wait_gather(p):
            hb = BB // 2
            for s in range(2):
                pltpu.make_async_copy(
                    tab_hbm.at[gidx_v.at[p, pl.ds(s * hb, hb)]],
                    prows_v.at[p, pl.ds(s * hb, hb)], gsem[p]).wait()

        def fire_out(u, q):
            l, bt = unit_lbt(u)
            for ct in range(CT):
                pltpu.async_copy(
                    y_v.at[q, pl.ds(ct * (8 * BB), 8 * BB)],
                    out_hbm.at[(l * CT + ct) * nbt + bt],
                    osem[q],
                )

        def wait_out(q):
            for ct in range(CT):
                pltpu.make_async_copy(
                    y_v.at[q, pl.ds(ct * (8 * BB), 8 * BB)],
                    out_hbm.at[0], osem[q]).wait()

        def transpose(p):
            q = p & 1
            def sb_body(sb, carry):
                g = lax.shift_right_logical(sb, 2)
                cb = lax.bitwise_and(sb, 3)
                bi_vec = iota + g * 16
                col0 = cb * 16
                svec = iota + (cb * (16 * BB) + g * 16)
                for j in range(0, 16, 8):
                    rots = [lax.bitwise_and(iota + (j + t), 15)
                            for t in range(8)]
                    vals = [plsc.load_gather(prows_v.at[p],
                                             [bi_vec, col0 + r])
                            for r in rots]
                    for r, v in zip(rots, vals):
                        plsc.store_scatter(y_v.at[q],
                                           [svec + lax.shift_left(r, 7)], v)
                return carry

            lax.fori_loop(0, (BB // 16) * (D // 16), sb_body, 0)

        def stage_b(u, p):
            wait_gather(p)
            wait_out(p & 1)
            transpose(p)
            fire_out(u, p & 1)

        for p in range(4):
            fire_idx(p, p)
        for p in range(4):
            stage_af(p, p)
            fire_idx(p + 4, p)
        fire_out(0, 0)
        fire_out(1, 1)

        def body4(kk, carry):
            u = 4 * kk
            for p in range(4):
                stage_b(u + p, p)
                stage_af(u + p + 4, p)
                fire_idx(u + p + 8, p)
            return carry

        lax.fori_loop(0, (upw - 8) // 4, body4, 0)

        for p in range(4):
            stage_b(upw - 8 + p, p)
            stage_af(upw - 4 + p, p)
        for p in range(4):
            stage_b(upw - 4 + p, p)
        wait_out(0)
        wait_out(1)

    return k(table2, ids_lm)


def kernel(ids, table):
    b, h = ids.shape
    v, d = table.shape
    assert d == D and b % BB == 0
    nbt = b // BB
    assert (h * nbt) % (8 * NW) == 0
    ids_lm = ids.T.reshape(-1).astype(jnp.int32)
    y = _gather(table, ids_lm, h, nbt)
    return (y.reshape(h, CT, nbt, 8, BB)
             .transpose(2, 4, 0, 1, 3)
             .reshape(b, h, D))
